# baseline (speedup 1.0000x reference)
import jax
import jax.numpy as jnp
from jax import lax
from jax.experimental import pallas as pl
from jax.experimental.pallas import tpu as pltpu
from jax.experimental.pallas import tpu_sc as plsc

B = 16384
D = 32
V = 1000000
NC = 2
NS = 16
L = 16
NW = NS
BPW = B // NW
SUB = 4
GRP = 2 * L
NSUB = GRP // SUB
NGRP = BPW // GRP


def _body(uidx_hbm, iidx_hbm, utab_hbm, itab_hbm, out_hbm,
          uidx_v, iidx_v, ublk, iblk, obuf, sem):
    wid = lax.axis_index("s")
    core = lax.axis_index("c")
    base = wid * BPW

    @pl.when(core == 0)
    def _diag_core0_only():
        _work(uidx_hbm, iidx_hbm, utab_hbm, itab_hbm, out_hbm,
              uidx_v, iidx_v, ublk, iblk, obuf, sem, wid, base)


def _work(uidx_hbm, iidx_hbm, utab_hbm, itab_hbm, out_hbm,
          uidx_v, iidx_v, ublk, iblk, obuf, sem, wid, base):
    pltpu.sync_copy(uidx_hbm.at[pl.ds(base, BPW)], uidx_v)
    pltpu.sync_copy(iidx_hbm.at[pl.ds(base, BPW)], iidx_v)

    lanes = lax.iota(jnp.int32, L)
    d_lo = lax.iota(jnp.int32, L)
    d_hi = d_lo + L

    def fire(sub, uvecs, ivecs, b):
        cps = []
        for k in range(SUB):
            jj = sub * SUB + k
            ru = uvecs[jj // L][jj % L]
            ri = ivecs[jj // L][jj % L]
            tcu = pl.multiple_of(
                lax.shift_right_logical(ru, 7) * jnp.int32(128), 128)
            tci = pl.multiple_of(
                lax.shift_right_logical(ri, 7) * jnp.int32(128), 128)
            cps.append(pltpu.async_copy(
                utab_hbm.at[:, pl.ds(tcu, 128)], ublk.at[b, k], sem))
            cps.append(pltpu.async_copy(
                itab_hbm.at[:, pl.ds(tci, 128)], iblk.at[b, k], sem))
        return cps

    def compute(sub, uvecs, ivecs, b, res):
        for k in range(SUB):
            jj = sub * SUB + k
            ucol = jnp.zeros((L,), jnp.int32) + lax.bitwise_and(
                uvecs[jj // L][jj % L], jnp.int32(127))
            icol = jnp.zeros((L,), jnp.int32) + lax.bitwise_and(
                ivecs[jj // L][jj % L], jnp.int32(127))
            ub = ublk.at[b, k]
            ib = iblk.at[b, k]
            u0 = plsc.load_gather(ub, [d_lo, ucol])
            u1 = plsc.load_gather(ub, [d_hi, ucol])
            v0 = plsc.load_gather(ib, [d_lo, icol])
            v1 = plsc.load_gather(ib, [d_hi, icol])
            p = u0 * v0 + u1 * v1
            s = lax.reduce_sum_p.bind(p, axes=(0,))
            res[jj // L] = jnp.where(lanes == jj % L, s, res[jj // L])
        return res

    def group(g, _):
        uvecs = [uidx_v[pl.ds(g * GRP, L)], uidx_v[pl.ds(g * GRP + L, L)]]
        ivecs = [iidx_v[pl.ds(g * GRP, L)], iidx_v[pl.ds(g * GRP + L, L)]]
        res = [jnp.zeros((L,), jnp.float32), jnp.zeros((L,), jnp.float32)]
        inflight = [fire(0, uvecs, ivecs, 0), fire(1, uvecs, ivecs, 1)]
        for sub in range(NSUB):
            if sub + 2 < NSUB:
                inflight.append(fire(sub + 2, uvecs, ivecs, (sub + 2) % 3))
            for cp in inflight.pop(0):
                cp.wait()
            res = compute(sub, uvecs, ivecs, sub % 3, res)
        obuf[pl.ds(g * GRP, L)] = res[0]
        obuf[pl.ds(g * GRP + L, L)] = res[1]
        return 0

    lax.fori_loop(0, NGRP, group, 0)

    pltpu.sync_copy(obuf, out_hbm.at[pl.ds(base, BPW)])


@jax.jit
def _mf_rating(user_indices, item_indices, user_table, item_table):
    mesh = plsc.VectorSubcoreMesh(
        core_axis_name="c", subcore_axis_name="s",
        num_cores=NC, num_subcores=NS)
    return pl.kernel(
        _body,
        out_type=jax.ShapeDtypeStruct((B,), jnp.float32),
        mesh=mesh,
        compiler_params=pltpu.CompilerParams(needs_layout_passes=False),
        scratch_types=[
            pltpu.VMEM((BPW,), jnp.int32),
            pltpu.VMEM((BPW,), jnp.int32),
            pltpu.VMEM((3, SUB, D, 128), jnp.float32),
            pltpu.VMEM((3, SUB, D, 128), jnp.float32),
            pltpu.VMEM((BPW,), jnp.float32),
            pltpu.SemaphoreType.DMA,
        ],
    )(user_indices, item_indices, user_table.T, item_table.T)


def kernel(user_indices, item_indices, user_table, item_table):
    return _mf_rating(user_indices, item_indices, user_table, item_table)


# speedup vs baseline: 1.7415x; 1.7415x over previous
import jax
import jax.numpy as jnp
from jax import lax
from jax.experimental import pallas as pl
from jax.experimental.pallas import tpu as pltpu
from jax.experimental.pallas import tpu_sc as plsc

B = 16384
D = 32
V = 1000000
NC = 2
NS = 16
L = 16
NW = NC * NS
BPW = B // NW
SUB = 4
GRP = 4 * L
NSUB = GRP // SUB
NGRP = BPW // GRP


def _body(uidx_hbm, iidx_hbm, utab_hbm, itab_hbm, out_hbm,
          uidx_v, iidx_v, ublk, iblk, obuf, sem):
    wid = lax.axis_index("s") * NC + lax.axis_index("c")
    base = wid * BPW

    pltpu.sync_copy(uidx_hbm.at[pl.ds(base, BPW)], uidx_v)
    pltpu.sync_copy(iidx_hbm.at[pl.ds(base, BPW)], iidx_v)

    lanes = lax.iota(jnp.int32, L)
    d_lo = lax.iota(jnp.int32, L)
    d_hi = d_lo + L

    def fire(sub, uvecs, ivecs, b):
        cps = []
        for k in range(SUB):
            jj = sub * SUB + k
            ru = uvecs[jj // L][jj % L]
            ri = ivecs[jj // L][jj % L]
            tcu = pl.multiple_of(
                lax.shift_right_logical(ru, 7) * jnp.int32(128), 128)
            tci = pl.multiple_of(
                lax.shift_right_logical(ri, 7) * jnp.int32(128), 128)
            cps.append(pltpu.async_copy(
                utab_hbm.at[:, pl.ds(tcu, 128)], ublk.at[b, k], sem))
            cps.append(pltpu.async_copy(
                itab_hbm.at[:, pl.ds(tci, 128)], iblk.at[b, k], sem))
        return cps

    def compute(sub, uvecs, ivecs, b, res):
        for k in range(SUB):
            jj = sub * SUB + k
            ucol = jnp.zeros((L,), jnp.int32) + lax.bitwise_and(
                uvecs[jj // L][jj % L], jnp.int32(127))
            icol = jnp.zeros((L,), jnp.int32) + lax.bitwise_and(
                ivecs[jj // L][jj % L], jnp.int32(127))
            ub = ublk.at[b, k]
            ib = iblk.at[b, k]
            u0 = plsc.load_gather(ub, [d_lo, ucol])
            u1 = plsc.load_gather(ub, [d_hi, ucol])
            v0 = plsc.load_gather(ib, [d_lo, icol])
            v1 = plsc.load_gather(ib, [d_hi, icol])
            p = u0 * v0 + u1 * v1
            s = lax.reduce_sum_p.bind(p, axes=(0,))
            res[jj // L] = jnp.where(lanes == jj % L, s, res[jj // L])
        return res

    def group(g, _):
        nv = GRP // L
        uvecs = [uidx_v[pl.ds(g * GRP + q * L, L)] for q in range(nv)]
        ivecs = [iidx_v[pl.ds(g * GRP + q * L, L)] for q in range(nv)]
        res = [jnp.zeros((L,), jnp.float32) for _ in range(nv)]
        inflight = [fire(0, uvecs, ivecs, 0), fire(1, uvecs, ivecs, 1)]
        for sub in range(NSUB):
            if sub + 2 < NSUB:
                inflight.append(fire(sub + 2, uvecs, ivecs, (sub + 2) % 3))
            for cp in inflight.pop(0):
                cp.wait()
            res = compute(sub, uvecs, ivecs, sub % 3, res)
        for q in range(nv):
            obuf[pl.ds(g * GRP + q * L, L)] = res[q]
        return 0

    lax.fori_loop(0, NGRP, group, 0)

    pltpu.sync_copy(obuf, out_hbm.at[pl.ds(base, BPW)])


@jax.jit
def _mf_rating(user_indices, item_indices, user_table, item_table):
    mesh = plsc.VectorSubcoreMesh(
        core_axis_name="c", subcore_axis_name="s",
        num_cores=NC, num_subcores=NS)
    return pl.kernel(
        _body,
        out_type=jax.ShapeDtypeStruct((B,), jnp.float32),
        mesh=mesh,
        compiler_params=pltpu.CompilerParams(needs_layout_passes=False),
        scratch_types=[
            pltpu.VMEM((BPW,), jnp.int32),
            pltpu.VMEM((BPW,), jnp.int32),
            pltpu.VMEM((3, SUB, D, 128), jnp.float32),
            pltpu.VMEM((3, SUB, D, 128), jnp.float32),
            pltpu.VMEM((BPW,), jnp.float32),
            pltpu.SemaphoreType.DMA,
        ],
    )(user_indices, item_indices, user_table.T, item_table.T)


def kernel(user_indices, item_indices, user_table, item_table):
    return _mf_rating(user_indices, item_indices, user_table, item_table)


# speedup vs baseline: 1.7532x; 1.0068x over previous
import jax
import jax.numpy as jnp
from jax import lax
from jax.experimental import pallas as pl
from jax.experimental.pallas import tpu as pltpu
from jax.experimental.pallas import tpu_sc as plsc

B = 16384
D = 32
V = 1000000
NC = 2
NS = 16
L = 16
NW = NC * NS
BPW = B // NW
SUB = 4
GRP = 8 * L
NSUB = GRP // SUB
NGRP = BPW // GRP


def _body(uidx_hbm, iidx_hbm, utab_hbm, itab_hbm, out_hbm,
          uidx_v, iidx_v, ublk, iblk, obuf, sem):
    wid = lax.axis_index("s") * NC + lax.axis_index("c")
    base = wid * BPW

    pltpu.sync_copy(uidx_hbm.at[pl.ds(base, BPW)], uidx_v)
    pltpu.sync_copy(iidx_hbm.at[pl.ds(base, BPW)], iidx_v)

    lanes = lax.iota(jnp.int32, L)
    d_lo = lax.iota(jnp.int32, L)
    d_hi = d_lo + L

    def fire(sub, uvecs, ivecs, b):
        cps = []
        for k in range(SUB):
            jj = sub * SUB + k
            ru = uvecs[jj // L][jj % L]
            ri = ivecs[jj // L][jj % L]
            tcu = pl.multiple_of(
                lax.shift_right_logical(ru, 7) * jnp.int32(128), 128)
            tci = pl.multiple_of(
                lax.shift_right_logical(ri, 7) * jnp.int32(128), 128)
            cps.append(pltpu.async_copy(
                utab_hbm.at[:, pl.ds(tcu, 128)], ublk.at[b, k], sem))
            cps.append(pltpu.async_copy(
                itab_hbm.at[:, pl.ds(tci, 128)], iblk.at[b, k], sem))
        return cps

    def compute(sub, uvecs, ivecs, b, res):
        for k in range(SUB):
            jj = sub * SUB + k
            ucol = jnp.zeros((L,), jnp.int32) + lax.bitwise_and(
                uvecs[jj // L][jj % L], jnp.int32(127))
            icol = jnp.zeros((L,), jnp.int32) + lax.bitwise_and(
                ivecs[jj // L][jj % L], jnp.int32(127))
            ub = ublk.at[b, k]
            ib = iblk.at[b, k]
            u0 = plsc.load_gather(ub, [d_lo, ucol])
            u1 = plsc.load_gather(ub, [d_hi, ucol])
            v0 = plsc.load_gather(ib, [d_lo, icol])
            v1 = plsc.load_gather(ib, [d_hi, icol])
            p = u0 * v0 + u1 * v1
            s = lax.reduce_sum_p.bind(p, axes=(0,))
            res[jj // L] = jnp.where(lanes == jj % L, s, res[jj // L])
        return res

    def group(g, _):
        nv = GRP // L
        uvecs = [uidx_v[pl.ds(g * GRP + q * L, L)] for q in range(nv)]
        ivecs = [iidx_v[pl.ds(g * GRP + q * L, L)] for q in range(nv)]
        res = [jnp.zeros((L,), jnp.float32) for _ in range(nv)]
        inflight = [fire(0, uvecs, ivecs, 0), fire(1, uvecs, ivecs, 1)]
        for sub in range(NSUB):
            if sub + 2 < NSUB:
                inflight.append(fire(sub + 2, uvecs, ivecs, (sub + 2) % 3))
            for cp in inflight.pop(0):
                cp.wait()
            res = compute(sub, uvecs, ivecs, sub % 3, res)
        for q in range(nv):
            obuf[pl.ds(g * GRP + q * L, L)] = res[q]
        return 0

    lax.fori_loop(0, NGRP, group, 0)

    pltpu.sync_copy(obuf, out_hbm.at[pl.ds(base, BPW)])


@jax.jit
def _mf_rating(user_indices, item_indices, user_table, item_table):
    mesh = plsc.VectorSubcoreMesh(
        core_axis_name="c", subcore_axis_name="s",
        num_cores=NC, num_subcores=NS)
    return pl.kernel(
        _body,
        out_type=jax.ShapeDtypeStruct((B,), jnp.float32),
        mesh=mesh,
        compiler_params=pltpu.CompilerParams(needs_layout_passes=False),
        scratch_types=[
            pltpu.VMEM((BPW,), jnp.int32),
            pltpu.VMEM((BPW,), jnp.int32),
            pltpu.VMEM((3, SUB, D, 128), jnp.float32),
            pltpu.VMEM((3, SUB, D, 128), jnp.float32),
            pltpu.VMEM((BPW,), jnp.float32),
            pltpu.SemaphoreType.DMA,
        ],
    )(user_indices, item_indices, user_table.T, item_table.T)


def kernel(user_indices, item_indices, user_table, item_table):
    return _mf_rating(user_indices, item_indices, user_table, item_table)
